# Initial kernel scaffold; baseline (speedup 1.0000x reference)
import jax
import jax.numpy as jnp
from jax.experimental import pallas as pl


def kernel(features, edge_index, W_self1, W_neigh1, b1, W_self2, W_neigh2, b2):
    raise NotImplementedError("write your pallas kernel here")


# speedup vs baseline: 3.5316x; 3.5316x over previous
import jax
import jax.numpy as jnp
from jax import lax
from jax.experimental import pallas as pl
from jax.experimental.pallas import tpu as pltpu
from jax.experimental.pallas import tpu_sc as plsc

N_NODES = 10000
N_PAD = 10240
NC, NS = 2, 16
NW = NC * NS
CHUNK = 128
NCH = 80
E_PAD = NW * NCH * CHUNK
IDXG = 8
RPT = N_PAD // NS
D = 128
DP = 48


def _make_sc_agg():
  mesh = plsc.VectorSubcoreMesh(core_axis_name="c", subcore_axis_name="s")
  scratch = [
      pltpu.VMEM((IDXG, CHUNK), jnp.int32),
      pltpu.VMEM((IDXG, CHUNK), jnp.int32),
      pltpu.VMEM((CHUNK, D), jnp.float32),
      pltpu.VMEM_SHARED((N_PAD, D), jnp.float32),
      pltpu.SemaphoreType.DMA,
  ]

  def body(table, srcb, dstb, zrows, agg_out, src_v, dst_v, rows_v, acc_sh,
           sem):
    c = lax.axis_index("c")
    s = lax.axis_index("s")
    wid = s * NC + c
    base = s * RPT
    pltpu.sync_copy(zrows, rows_v)
    for k in range(RPT // CHUNK):
      pltpu.sync_copy(rows_v, acc_sh.at[pl.ds(base + k * CHUNK, CHUNK)])
    plsc.subcore_barrier()

    def group(g, carry):
      pltpu.sync_copy(srcb.at[wid, pl.ds(g * IDXG, IDXG)], src_v)
      pltpu.sync_copy(dstb.at[wid, pl.ds(g * IDXG, IDXG)], dst_v)

      def step(j, carry2):
        pltpu.async_copy(table.at[src_v.at[j]], rows_v, sem).wait()
        pltpu.sync_copy(rows_v, acc_sh.at[dst_v.at[j]], add=True)
        return carry2

      lax.fori_loop(0, IDXG, step, 0)
      return carry

    lax.fori_loop(0, NCH // IDXG, group, 0)
    plsc.subcore_barrier()
    for k in range(RPT // CHUNK):
      off = base + k * CHUNK
      pltpu.sync_copy(acc_sh.at[pl.ds(off, CHUNK)], rows_v)
      pltpu.sync_copy(rows_v, agg_out.at[c, pl.ds(off, CHUNK)])

  return pl.kernel(
      body,
      out_type=jax.ShapeDtypeStruct((NC, N_PAD, D), jnp.float32),
      mesh=mesh, scratch_types=scratch)


def _make_sc_deg():
  mesh = plsc.VectorSubcoreMesh(core_axis_name="c", subcore_axis_name="s")
  scratch = [
      pltpu.VMEM((IDXG, CHUNK), jnp.int32),
      pltpu.VMEM((CHUNK, D), jnp.float32),
      pltpu.VMEM_SHARED((N_PAD, D), jnp.float32),
  ]

  def body(dstb, zrows, ones, deg_out, dst_v, ones_v, deg_sh):
    c = lax.axis_index("c")
    s = lax.axis_index("s")
    wid = s * NC + c
    base = s * RPT
    pltpu.sync_copy(zrows, ones_v)
    for k in range(RPT // CHUNK):
      pltpu.sync_copy(ones_v, deg_sh.at[pl.ds(base + k * CHUNK, CHUNK)])
    pltpu.sync_copy(ones, ones_v)
    plsc.subcore_barrier()

    def group(g, carry):
      pltpu.sync_copy(dstb.at[wid, pl.ds(g * IDXG, IDXG)], dst_v)

      def step(j, carry2):
        pltpu.sync_copy(ones_v, deg_sh.at[dst_v.at[j]], add=True)
        return carry2

      lax.fori_loop(0, IDXG, step, 0)
      return carry

    lax.fori_loop(0, NCH // IDXG, group, 0)
    plsc.subcore_barrier()
    for k in range(RPT // CHUNK):
      off = base + k * CHUNK
      pltpu.sync_copy(deg_sh.at[pl.ds(off, CHUNK)], ones_v)
      pltpu.sync_copy(ones_v, deg_out.at[c, pl.ds(off, CHUNK)])

  return pl.kernel(
      body,
      out_type=jax.ShapeDtypeStruct((NC, N_PAD, D), jnp.float32),
      mesh=mesh, scratch_types=scratch)


_sc_agg = _make_sc_agg()
_sc_deg = _make_sc_deg()


def _tc_layer(x, a, dg, ws1, wn1, b1r, ws2, wn2, b2r):
  BR = 256

  def body(x_ref, a_ref, dg_ref, ws1_r, wn1_r, b1_ref, ws2_r, wn2_r, b2_ref,
           p_ref, os_ref):
    deg = dg_ref[0] + dg_ref[1]
    inv = 1.0 / jnp.maximum(deg, 1.0)
    mean = (a_ref[0] + a_ref[1]) * inv[:, 0:1]
    h = jnp.dot(x_ref[...], ws1_r[...], preferred_element_type=jnp.float32)
    h = h + jnp.dot(mean, wn1_r[...], preferred_element_type=jnp.float32)
    h = jnp.maximum(h + b1_ref[...], 0.0)
    p_ref[...] = jnp.dot(h, wn2_r[...], preferred_element_type=jnp.float32)
    os_ref[...] = (jnp.dot(h, ws2_r[...], preferred_element_type=jnp.float32)
                   + b2_ref[...])

  return pl.pallas_call(
      body,
      grid=(N_PAD // BR,),
      in_specs=[
          pl.BlockSpec((BR, 128), lambda i: (i, 0)),
          pl.BlockSpec((NC, BR, 128), lambda i: (0, i, 0)),
          pl.BlockSpec((NC, BR, 128), lambda i: (0, i, 0)),
          pl.BlockSpec((128, 256), lambda i: (0, 0)),
          pl.BlockSpec((128, 256), lambda i: (0, 0)),
          pl.BlockSpec((1, 256), lambda i: (0, 0)),
          pl.BlockSpec((256, DP), lambda i: (0, 0)),
          pl.BlockSpec((256, 128), lambda i: (0, 0)),
          pl.BlockSpec((1, DP), lambda i: (0, 0)),
      ],
      out_specs=[
          pl.BlockSpec((BR, 128), lambda i: (i, 0)),
          pl.BlockSpec((BR, DP), lambda i: (i, 0)),
      ],
      out_shape=[
          jax.ShapeDtypeStruct((N_PAD, 128), jnp.float32),
          jax.ShapeDtypeStruct((N_PAD, DP), jnp.float32),
      ],
  )(x, a, dg, ws1, wn1, b1r, ws2, wn2, b2r)


def _tc_combine(oself, q, dg):
  BR = 1024

  def body(os_ref, q_ref, dg_ref, out_ref):
    deg = dg_ref[0] + dg_ref[1]
    inv = 1.0 / jnp.maximum(deg, 1.0)
    qs = q_ref[0] + q_ref[1]
    out_ref[...] = os_ref[...] + qs[:, :DP] * inv[:, 0:1]

  return pl.pallas_call(
      body,
      grid=(N_PAD // BR,),
      in_specs=[
          pl.BlockSpec((BR, DP), lambda i: (i, 0)),
          pl.BlockSpec((NC, BR, 128), lambda i: (0, i, 0)),
          pl.BlockSpec((NC, BR, 128), lambda i: (0, i, 0)),
      ],
      out_specs=pl.BlockSpec((BR, DP), lambda i: (i, 0)),
      out_shape=jax.ShapeDtypeStruct((N_PAD, DP), jnp.float32),
  )(oself, q, dg)


def kernel(features, edge_index, W_self1, W_neigh1, b1, W_self2, W_neigh2,
           b2):
  f32 = jnp.float32
  x = jnp.pad(features, ((0, N_PAD - N_NODES), (0, 0)))
  src = edge_index[0]
  dst = edge_index[1]
  pad_e = E_PAD - src.shape[0]
  srcb = jnp.concatenate(
      [src, jnp.zeros((pad_e,), jnp.int32)]).reshape(NW, NCH, CHUNK)
  dstb = jnp.concatenate(
      [dst, jnp.full((pad_e,), N_PAD - 1, jnp.int32)]).reshape(NW, NCH, CHUNK)
  zrows = jnp.zeros((CHUNK, 128), f32)
  ones = jnp.ones((CHUNK, 128), f32)

  degp = _sc_deg(dstb, zrows, ones)
  zrows2, degp = lax.optimization_barrier((zrows, degp))
  agg1 = _sc_agg(x, srcb, dstb, zrows2)

  w2s = jnp.pad(W_self2, ((0, 0), (0, DP - W_self2.shape[1])))
  w2n = jnp.pad(W_neigh2, ((0, 0), (0, 128 - W_neigh2.shape[1])))
  b2r = jnp.pad(b2, (0, DP - b2.shape[0])).reshape(1, DP)
  b1r = b1.reshape(1, 256)
  p, oself = _tc_layer(x, agg1, degp, W_self1, W_neigh1, b1r, w2s, w2n, b2r)

  q = _sc_agg(p, srcb, dstb, zrows)

  out = _tc_combine(oself, q, degp)
  return out[:N_NODES, :W_self2.shape[1]]
